# baseline (speedup 1.0000x reference)
import functools

import jax
import jax.numpy as jnp
from jax import lax
from jax.experimental import pallas as pl
from jax.experimental.pallas import tpu as pltpu
from jax.experimental.pallas import tpu_sc as plsc


def _proj_body(tt_ref, w_ref, out_ref):
    out_ref[...] = lax.dot_general(
        tt_ref[...],
        w_ref[...],
        (((0,), (0,)), ((), ())),
        preferred_element_type=jnp.float32,
    )


def _tc_project(tableT, WT, BW=4096):
    D, V = tableT.shape
    E = WT.shape[1]
    grid = (pl.cdiv(V, BW),)
    return pl.pallas_call(
        _proj_body,
        grid=grid,
        in_specs=[
            pl.BlockSpec((D, BW), lambda g: (0, g)),
            pl.BlockSpec((D, E), lambda g: (0, 0)),
        ],
        out_specs=pl.BlockSpec((BW, E), lambda g: (g, 0)),
        out_shape=jax.ShapeDtypeStruct((V, E), jnp.float32),
        compiler_params=pltpu.CompilerParams(
            fuse_transposed_lhs_in_matmul=True
        ),
    )(tableT, WT)


def _sc_gather(N, V, E):
    NW = 32
    n_w = N // NW
    C = 256
    NB = 3
    n_chunks = n_w // C
    assert n_chunks >= NB
    mesh = plsc.VectorSubcoreMesh(core_axis_name="c", subcore_axis_name="s")

    @functools.partial(
        pl.kernel,
        mesh=mesh,
        compiler_params=pltpu.CompilerParams(use_tc_tiling_on_sc=False),
        out_type=jax.ShapeDtypeStruct((N, E), jnp.float32),
        scratch_types=[
            pltpu.VMEM((n_w,), jnp.int32),
            pltpu.VMEM((NB, C, E), jnp.float32),
            pltpu.SemaphoreType.DMA,
            pltpu.SemaphoreType.DMA,
        ],
    )
    def gather_kernel(idx_hbm, ptab_hbm, out_hbm, idx_v, rows_v, gsem, wsem):
        wid = lax.axis_index("s") * 2 + lax.axis_index("c")
        base = wid * n_w

        pltpu.sync_copy(idx_hbm.at[pl.ds(base, n_w)], idx_v)

        def gather_start(i):
            pltpu.async_copy(
                ptab_hbm.at[idx_v.at[pl.ds(i * C, C)]], rows_v.at[i % NB], gsem
            )

        def gather_wait(i):
            pltpu.make_async_copy(
                ptab_hbm.at[idx_v.at[pl.ds(i * C, C)]], rows_v.at[i % NB], gsem
            ).wait()

        def write_start(i):
            pltpu.async_copy(
                rows_v.at[i % NB], out_hbm.at[pl.ds(base + i * C, C)], wsem
            )

        def write_wait(i):
            pltpu.make_async_copy(
                rows_v.at[i % NB], out_hbm.at[pl.ds(base + i * C, C)], wsem
            ).wait()

        gather_start(0)
        gather_start(1)

        def body(i, carry):
            gather_wait(i)
            write_start(i)

            @pl.when(i + 2 < n_chunks)
            def _():
                @pl.when(i >= 1)
                def _():
                    write_wait(i - 1)

                gather_start(i + 2)

            return carry

        lax.fori_loop(0, n_chunks, body, 0)
        write_wait(n_chunks - 3)
        write_wait(n_chunks - 2)
        write_wait(n_chunks - 1)

    return gather_kernel


def kernel(x, table, W_up):
    B, L = x.shape
    V, D = table.shape
    E = W_up.shape[0]
    N = B * L

    idxT = jnp.transpose(x).reshape(N)

    tableT = jnp.transpose(table)
    WT = jnp.transpose(W_up)
    ptab = _tc_project(tableT, WT)

    outT = _sc_gather(N, V, E)(idxT, ptab)

    return outT.reshape(L, B, E).transpose(1, 0, 2)
